# Initial kernel scaffold; baseline (speedup 1.0000x reference)
import jax
import jax.numpy as jnp
from jax.experimental import pallas as pl


def kernel(x1, edge_index1, edge_attr1, x2, edge_index2, edge_attr2, W0, b0, Wf, bf, root, b_ecc, Wg, a_self, a_neigh, b_gat, Wfc, bfc, Wreg, breg):
    raise NotImplementedError("write your pallas kernel here")


# speedup vs baseline: 2.2119x; 2.2119x over previous
import functools

import jax
import jax.numpy as jnp
from jax import lax
from jax.experimental import pallas as pl
from jax.experimental.pallas import tpu as pltpu
from jax.experimental.pallas import tpu_sc as plsc

N = 10000
E = 160000
NN = 2 * N
F_IN = 128
F_EDGE = 16
F_OUT = 64
H_KN = 32

CHK = 2048
CPC = 80
EPC = CPC * CHK
EEP = 2 * EPC
PADE = EPC - E
HCH = CPC // 2
CPT = 8
ACCF = (N + 8) * CPT

CH = 128
NCHG = EEP // CH
GIT = NCHG // 32

EB = 512
NEBP = EEP // EB
NB = 2000
NNB = NN // NB

_mesh = lambda: plsc.VectorSubcoreMesh(core_axis_name="c", subcore_axis_name="s")
_cp = pltpu.CompilerParams(needs_layout_passes=False)


@functools.partial(
    pl.kernel,
    out_type=jax.ShapeDtypeStruct((EEP, F_IN), jnp.float32),
    mesh=_mesh(),
    compiler_params=_cp,
    scratch_types=[
        pltpu.VMEM((CH,), jnp.int32),
        pltpu.VMEM((CH, F_IN), jnp.float32),
        pltpu.SemaphoreType.DMA,
    ],
)
def _gather_k(table, idx, out, idx_v, rows_v, sem):
    w = lax.axis_index("s") * 2 + lax.axis_index("c")

    def body(i, carry):
        base = pl.multiple_of((i * 32 + w) * CH, CH)
        pltpu.sync_copy(idx.at[pl.ds(base, CH)], idx_v)
        pltpu.async_copy(table.at[idx_v], rows_v, sem).wait()
        pltpu.sync_copy(rows_v, out.at[pl.ds(base, CH)])
        return carry

    lax.fori_loop(0, GIT, body, 0)


def _make_scatter(ncg, halved):
    W = ncg * CPT
    oshape = (2, 2, ncg, N * CPT) if halved else (2, ncg, N * CPT)

    @functools.partial(
        pl.kernel,
        out_type=jax.ShapeDtypeStruct(oshape, jnp.float32),
        mesh=_mesh(),
        compiler_params=_cp,
        scratch_types=[
            pltpu.VMEM((CHK,), jnp.int32),
            pltpu.VMEM((CPT, CHK), jnp.float32),
            pltpu.VMEM((ACCF,), jnp.float32),
        ],
    )
    def scatter_k(payT, dstl, zeros, out, idx_v, rows_v, acc):
        c = lax.axis_index("c")
        s = lax.axis_index("s")
        if halved:
            h = s // ncg
            g = s % ncg
            nchunks = HCH
        else:
            h = None
            g = s
            nchunks = CPC
        pltpu.sync_copy(zeros, acc)
        gro = pl.multiple_of(g * CPT, CPT)
        iota16 = lax.iota(jnp.int32, 16)

        def body(i, carry):
            cid = (h * HCH + i) if halved else i
            base = pl.multiple_of(c * EPC + cid * CHK, CHK)
            pltpu.sync_copy(dstl.at[pl.ds(base, CHK)], idx_v)
            pltpu.sync_copy(payT.at[pl.ds(gro, CPT), pl.ds(base, CHK)], rows_v)
            for j in range(CHK // 16):
                dv = idx_v[pl.ds(j * 16, 16)]
                abase = dv * CPT
                ei = j * 16 + iota16
                for k in range(CPT):
                    vals = plsc.load_gather(
                        rows_v, [jnp.full((16,), k, jnp.int32), ei])
                    plsc.addupdate_scatter(acc, [abase + k], vals)
            return carry

        lax.fori_loop(0, nchunks, body, 0)
        dst_ref = out.at[c].at[h].at[g] if halved else out.at[c].at[g]
        pltpu.sync_copy(acc.at[pl.ds(0, N * CPT)], dst_ref)

    return scatter_k


@functools.partial(
    pl.kernel,
    out_type=jax.ShapeDtypeStruct((EEP,), jnp.float32),
    mesh=_mesh(),
    compiler_params=_cp,
    scratch_types=[
        pltpu.VMEM((NN,), jnp.float32),
        pltpu.VMEM((NN,), jnp.float32),
        pltpu.VMEM((NN,), jnp.float32),
        pltpu.VMEM((CH,), jnp.int32),
        pltpu.VMEM((CH,), jnp.int32),
        pltpu.VMEM((CH,), jnp.float32),
    ],
)
def _gat_logits_k(a_s, a_n, sshift, src, dst, out,
                  as_v, an_v, s_v, sidx, didx, exb):
    w = lax.axis_index("s") * 2 + lax.axis_index("c")
    pltpu.sync_copy(a_s, as_v)
    pltpu.sync_copy(a_n, an_v)
    pltpu.sync_copy(sshift, s_v)

    def body(i, carry):
        base = pl.multiple_of((i * 32 + w) * CH, CH)
        pltpu.sync_copy(src.at[pl.ds(base, CH)], sidx)
        pltpu.sync_copy(dst.at[pl.ds(base, CH)], didx)
        for g in range(CH // 16):
            sv = sidx[pl.ds(g * 16, 16)]
            dv = didx[pl.ds(g * 16, 16)]
            asd = plsc.load_gather(as_v, [dv])
            ans = plsc.load_gather(an_v, [sv])
            sd = plsc.load_gather(s_v, [dv])
            e = asd + ans
            e = jnp.where(e > 0, e, 0.2 * e)
            exb[pl.ds(g * 16, 16)] = jnp.exp(e - sd)
        pltpu.sync_copy(exb, out.at[pl.ds(base, CH)])
        return carry

    lax.fori_loop(0, GIT, body, 0)


def _ecc_msg_body(xs_ref, ea_ref, w0_ref, b0_ref, kaug_ref, out_ref):
    ea = ea_ref[...]
    h = jnp.maximum(jnp.dot(ea, w0_ref[...],
                            preferred_element_type=jnp.float32) + b0_ref[...], 0.0)
    p = jnp.dot(xs_ref[...].astype(jnp.bfloat16), kaug_ref[...],
                preferred_element_type=jnp.float32)
    msg = p[:, H_KN * F_OUT:(H_KN + 1) * F_OUT]
    for j in range(H_KN):
        msg = msg + h[:, j:j + 1] * p[:, j * F_OUT:(j + 1) * F_OUT]
    out_ref[...] = msg.T


def _ecc_msg(xs_f32, ea, w0, b0, kaug):
    return pl.pallas_call(
        _ecc_msg_body,
        grid=(NEBP,),
        in_specs=[
            pl.BlockSpec((EB, F_IN), lambda i: (i, 0)),
            pl.BlockSpec((EB, F_EDGE), lambda i: (i, 0)),
            pl.BlockSpec((F_EDGE, H_KN), lambda i: (0, 0)),
            pl.BlockSpec((1, H_KN), lambda i: (0, 0)),
            pl.BlockSpec((F_IN, (H_KN + 1) * F_OUT), lambda i: (0, 0)),
        ],
        out_specs=pl.BlockSpec((F_OUT, EB), lambda i: (0, i)),
        out_shape=jax.ShapeDtypeStruct((F_OUT, EEP), jnp.float32),
    )(xs_f32, ea, w0, b0, kaug)


def _node_body(agg0_ref, agg1_ref, x_ref, root_ref, becc_ref, wg_ref,
               aself_ref, aneigh_ref, feat_ref, as_ref, an_ref):
    agg = agg0_ref[...] + agg1_ref[...]
    x1 = jnp.maximum(agg + jnp.dot(x_ref[...], root_ref[...],
                                   preferred_element_type=jnp.float32)
                     + becc_ref[...], 0.0)
    feat = jnp.dot(x1, wg_ref[...], preferred_element_type=jnp.float32)
    a_s = jnp.dot(feat, aself_ref[...], preferred_element_type=jnp.float32)
    a_n = jnp.dot(feat, aneigh_ref[...], preferred_element_type=jnp.float32)
    feat_ref[:, :F_OUT] = feat
    feat_ref[:, F_OUT:F_OUT + 1] = jnp.ones((NB, 1), jnp.float32)
    feat_ref[:, F_OUT + 1:] = jnp.zeros((NB, F_IN - F_OUT - 1), jnp.float32)
    as_ref[...] = a_s
    an_ref[...] = a_n


def _node_stage(agg0, agg1, x_cat, root, b_ecc, wg, a_self, a_neigh):
    return pl.pallas_call(
        _node_body,
        grid=(NNB,),
        in_specs=[pl.BlockSpec((NB, F_OUT), lambda i: (i, 0)),
                  pl.BlockSpec((NB, F_OUT), lambda i: (i, 0)),
                  pl.BlockSpec((NB, F_IN), lambda i: (i, 0)),
                  pl.BlockSpec((F_IN, F_OUT), lambda i: (0, 0)),
                  pl.BlockSpec((1, F_OUT), lambda i: (0, 0)),
                  pl.BlockSpec((F_OUT, F_OUT), lambda i: (0, 0)),
                  pl.BlockSpec((F_OUT, 1), lambda i: (0, 0)),
                  pl.BlockSpec((F_OUT, 1), lambda i: (0, 0))],
        out_specs=[pl.BlockSpec((NB, F_IN), lambda i: (i, 0)),
                   pl.BlockSpec((NB, 1), lambda i: (i, 0)),
                   pl.BlockSpec((NB, 1), lambda i: (i, 0))],
        out_shape=[jax.ShapeDtypeStruct((NN, F_IN), jnp.float32),
                   jax.ShapeDtypeStruct((NN, 1), jnp.float32),
                   jax.ShapeDtypeStruct((NN, 1), jnp.float32)],
    )(agg0, agg1, x_cat, root, b_ecc, wg, a_self, a_neigh)


def _shift_body(as_ref, an_ref, s_ref):
    a_n = an_ref[...]
    amax1 = jnp.max(a_n[:N])
    amax2 = jnp.max(a_n[N:])
    row = lax.broadcasted_iota(jnp.int32, (NN, 1), 0)
    sarg = as_ref[...] + jnp.where(row < N, amax1, amax2)
    s_ref[...] = jnp.where(sarg > 0, sarg, 0.2 * sarg)


def _shift_stage(a_s, a_n):
    return pl.pallas_call(
        _shift_body,
        in_specs=[pl.BlockSpec((NN, 1), lambda: (0, 0)),
                  pl.BlockSpec((NN, 1), lambda: (0, 0))],
        out_specs=pl.BlockSpec((NN, 1), lambda: (0, 0)),
        out_shape=jax.ShapeDtypeStruct((NN, 1), jnp.float32),
    )(a_s, a_n)


def _gat_payload_body(ex_ref, frows_ref, out_ref):
    out_ref[...] = (ex_ref[...] * frows_ref[...]).T


def _gat_payload(ex_col, frows):
    return pl.pallas_call(
        _gat_payload_body,
        grid=(NEBP,),
        in_specs=[pl.BlockSpec((EB, 1), lambda i: (i, 0)),
                  pl.BlockSpec((EB, F_IN), lambda i: (i, 0))],
        out_specs=pl.BlockSpec((F_IN, EB), lambda i: (0, i)),
        out_shape=jax.ShapeDtypeStruct((F_IN, EEP), jnp.float32),
    )(ex_col, frows)


def _final_body(o_ref, bgat_ref, wfc_ref, bfc_ref, wreg_ref, breg_ref, out_ref):
    o = o_ref[...]
    den = o[:, F_OUT:F_OUT + 1]
    den = jnp.where(den > 0, den, 1.0)
    x2 = jnp.maximum(o[:, :F_OUT] / den + bgat_ref[...], 0.0)
    p1 = jnp.mean(x2[:N], axis=0, keepdims=True)
    p2 = jnp.mean(x2[N:], axis=0, keepdims=True)
    z1 = jnp.maximum(jnp.dot(p1, wfc_ref[...],
                             preferred_element_type=jnp.float32) + bfc_ref[...], 0.0)
    z2 = jnp.maximum(jnp.dot(p2, wfc_ref[...],
                             preferred_element_type=jnp.float32) + bfc_ref[...], 0.0)
    d = jnp.abs(z1 - z2)
    out_ref[...] = jax.nn.sigmoid(
        jnp.dot(d, wreg_ref[...], preferred_element_type=jnp.float32)
        + breg_ref[...])


def _final_stage(o, b_gat, wfc, bfc, wreg, breg):
    return pl.pallas_call(
        _final_body,
        in_specs=[pl.BlockSpec((NN, F_IN), lambda: (0, 0)),
                  pl.BlockSpec((1, F_OUT), lambda: (0, 0)),
                  pl.BlockSpec((F_OUT, 32), lambda: (0, 0)),
                  pl.BlockSpec((1, 32), lambda: (0, 0)),
                  pl.BlockSpec((32, 1), lambda: (0, 0)),
                  pl.BlockSpec((1, 1), lambda: (0, 0))],
        out_specs=pl.BlockSpec((1, 1), lambda: (0, 0)),
        out_shape=jax.ShapeDtypeStruct((1, 1), jnp.float32),
    )(o, b_gat, wfc, bfc, wreg, breg)


def _unscatter(parts, ncg):
    return parts.reshape(2, ncg, N, CPT).transpose(0, 2, 1, 3).reshape(
        NN, ncg * CPT)


def kernel(x1, edge_index1, edge_attr1, x2, edge_index2, edge_attr2,
           W0, b0, Wf, bf, root, b_ecc, Wg, a_self, a_neigh, b_gat,
           Wfc, bfc, Wreg, breg):
    x_cat = jnp.concatenate([x1, x2], axis=0)
    padi = jnp.zeros((PADE,), jnp.int32)
    padt = jnp.full((PADE,), N, jnp.int32)
    pada = jnp.zeros((PADE, F_EDGE), jnp.float32)
    src_g = jnp.concatenate([edge_index1[0], padi,
                             edge_index2[0] + N, padi])
    dst_g = jnp.concatenate([edge_index1[1], padi,
                             edge_index2[1] + N, padi])
    dst_l = jnp.concatenate([edge_index1[1], padt,
                             edge_index2[1], padt])
    ea = jnp.concatenate([edge_attr1, pada, edge_attr2, pada], axis=0)

    kflat = Wf.reshape(H_KN, F_IN, F_OUT).transpose(1, 0, 2).reshape(
        F_IN, H_KN * F_OUT)
    kaug = jnp.concatenate([kflat, bf.reshape(F_IN, F_OUT)], axis=1)
    kaug_bf = kaug.astype(jnp.bfloat16)
    zeros_acc = jnp.zeros((ACCF,), jnp.float32)

    xs = _gather_k(x_cat, src_g)
    msgT = _ecc_msg(xs, ea, W0, b0.reshape(1, H_KN), kaug_bf)
    aggp = _make_scatter(F_OUT // CPT, True)(msgT, dst_l, zeros_acc)
    agg0 = _unscatter(aggp[:, 0], F_OUT // CPT)
    agg1 = _unscatter(aggp[:, 1], F_OUT // CPT)

    feat, a_s, a_n = _node_stage(
        agg0, agg1, x_cat, root, b_ecc.reshape(1, F_OUT),
        Wg, a_self, a_neigh)
    sshift = _shift_stage(a_s, a_n)

    ex = _gat_logits_k(a_s.reshape(NN), a_n.reshape(NN), sshift.reshape(NN),
                       src_g, dst_g)
    frows = _gather_k(feat, src_g)
    prodT = _gat_payload(ex.reshape(EEP, 1), frows)
    outp = _make_scatter(F_IN // CPT, False)(prodT, dst_l, zeros_acc)
    o = _unscatter(outp, F_IN // CPT)

    res = _final_stage(o, b_gat.reshape(1, F_OUT),
                       Wfc, bfc.reshape(1, 32), Wreg, breg.reshape(1, 1))
    return res.reshape(1)
